# Initial kernel scaffold; baseline (speedup 1.0000x reference)
import jax
import jax.numpy as jnp
from jax.experimental import pallas as pl


def kernel(x, conv1, bn1_scale, bn1_shift, l0b0_conv1, l0b0_bn1_scale, l0b0_bn1_shift, l0b0_conv2, l0b0_bn2_scale, l0b0_bn2_shift, l0b1_conv1, l0b1_bn1_scale, l0b1_bn1_shift, l0b1_conv2, l0b1_bn2_scale, l0b1_bn2_shift, l1b0_conv1, l1b0_bn1_scale, l1b0_bn1_shift, l1b0_conv2, l1b0_bn2_scale, l1b0_bn2_shift, l1b0_down_conv, l1b0_down_bn_scale, l1b0_down_bn_shift, l1b1_conv1, l1b1_bn1_scale, l1b1_bn1_shift, l1b1_conv2, l1b1_bn2_scale, l1b1_bn2_shift, l2b0_conv1, l2b0_bn1_scale, l2b0_bn1_shift, l2b0_conv2, l2b0_bn2_scale, l2b0_bn2_shift, l2b0_down_conv, l2b0_down_bn_scale, l2b0_down_bn_shift, l2b1_conv1, l2b1_bn1_scale, l2b1_bn1_shift, l2b1_conv2, l2b1_bn2_scale, l2b1_bn2_shift, l3b0_conv1, l3b0_bn1_scale, l3b0_bn1_shift, l3b0_conv2, l3b0_bn2_scale, l3b0_bn2_shift, l3b0_down_conv, l3b0_down_bn_scale, l3b0_down_bn_shift, l3b1_conv1, l3b1_bn1_scale, l3b1_bn1_shift, l3b1_conv2, l3b1_bn2_scale, l3b1_bn2_shift, cls_w, cls_b):
    raise NotImplementedError("write your pallas kernel here")


# speedup vs baseline: 1.1103x; 1.1103x over previous
import functools
import math

import jax
import jax.numpy as jnp
from jax.experimental import pallas as pl
from jax.experimental.pallas import tpu as pltpu


def _ru(x, m):
    return (x + m - 1) // m * m


def _mm_body(a_ref, b_ref, s_ref, t_ref, o_ref, *, relu):
    y = jnp.dot(a_ref[...], b_ref[...], preferred_element_type=jnp.float32)
    y = y * s_ref[...] + t_ref[...]
    if relu:
        y = jnp.maximum(y, 0.0)
    o_ref[...] = y.astype(o_ref.dtype)


def _mm_affine(a, b, scale, shift, *, relu, tm, out_dtype):
    M, K = a.shape
    _, N = b.shape
    Mp, Kp, Np = _ru(M, tm), _ru(K, 128), _ru(N, 64)
    a_p = jnp.pad(a.astype(jnp.bfloat16), ((0, Mp - M), (0, Kp - K)))
    b_p = jnp.pad(b.astype(jnp.bfloat16), ((0, Kp - K), (0, Np - N)))
    s_p = jnp.pad(scale.astype(jnp.float32).reshape(1, N), ((0, 0), (0, Np - N)))
    t_p = jnp.pad(shift.astype(jnp.float32).reshape(1, N), ((0, 0), (0, Np - N)))
    out = pl.pallas_call(
        functools.partial(_mm_body, relu=relu),
        out_shape=jax.ShapeDtypeStruct((Mp, Np), out_dtype),
        grid=(Mp // tm,),
        in_specs=[pl.BlockSpec((tm, Kp), lambda i: (i, 0)),
                  pl.BlockSpec((Kp, Np), lambda i: (0, 0)),
                  pl.BlockSpec((1, Np), lambda i: (0, 0)),
                  pl.BlockSpec((1, Np), lambda i: (0, 0))],
        out_specs=pl.BlockSpec((tm, Np), lambda i: (i, 0)),
        compiler_params=pltpu.CompilerParams(
            dimension_semantics=("parallel",)),
    )(a_p, b_p, s_p, t_p)
    if (Mp, Np) != (M, N):
        out = out[:M, :N]
    return out


def _conv_body(*refs, taps, relu, has_res, has_down):
    it = iter(refs)
    x_ref, w_ref, sc_ref, sh_ref = next(it), next(it), next(it), next(it)
    if has_down:
        dw_ref, dsc_ref, dsh_ref = next(it), next(it), next(it)
    if has_res:
        r_ref = next(it)
    o_ref = next(it)
    if has_down:
        d_ref = next(it)
    acc_ref = next(it)
    L = o_ref.shape[1]
    for t, (p, off) in enumerate(taps):
        part = jnp.dot(x_ref[0, p, off:off + L, :], w_ref[t],
                       preferred_element_type=jnp.float32)
        if t == 0:
            acc_ref[...] = part
        else:
            acc_ref[...] += part
    y = acc_ref[...] * sc_ref[...] + sh_ref[...]
    if has_res:
        y = y + r_ref[0].astype(jnp.float32)
    if relu:
        y = jnp.maximum(y, 0.0)
    o_ref[0] = y.astype(o_ref.dtype)
    if has_down:
        z = jnp.dot(x_ref[0, 3, 0:L, :], dw_ref[0],
                    preferred_element_type=jnp.float32)
        d_ref[0] = (z * dsc_ref[...] + dsh_ref[...]).astype(d_ref.dtype)


def _split_phases(xp, s, flen):
    B, Hp, Wp, C = xp.shape
    Hr, Wr = _ru(Hp, s), _ru(Wp, s)
    if (Hr, Wr) != (Hp, Wp):
        xp = jnp.pad(xp, ((0, 0), (0, Hr - Hp), (0, Wr - Wp), (0, 0)))
    Hq, Wq = Hr // s, Wr // s
    if s == 1:
        ph = xp.reshape(B, 1, Hq * Wq, C)
    else:
        ph = xp.reshape(B, Hq, s, Wq, s, C)
        ph = ph.transpose(0, 2, 4, 1, 3, 5).reshape(B, s * s, Hq * Wq, C)
    if flen > Hq * Wq:
        ph = jnp.pad(ph, ((0, 0), (0, 0), (0, flen - Hq * Wq), (0, 0)))
    return ph


def _conv_bn_act(x, w, scale, shift, *, k, stride, pad, relu=True,
                 residual=None, down=None):
    B, H, W, Cin = x.shape
    T, _, Cout = w.shape
    s = stride
    Hp, Wp = H + 2 * pad, W + 2 * pad
    Ho, Wo = (Hp - k) // s + 1, (Wp - k) // s + 1
    Hq, Wq = _ru(Hp, s) // s, _ru(Wp, s) // s
    L = _ru(Ho * Wq, 8)
    taps = tuple(((di % s) * s + (dj % s), (di // s) * Wq + (dj // s))
                 for di in range(k) for dj in range(k))
    flen = max(max(off for _, off in taps) + L, Hq * Wq)
    P = s * s
    has_down, has_res = down is not None, residual is not None
    if has_down:
        assert s == 2 and pad == 1
    xp = jnp.pad(x.astype(jnp.bfloat16),
                 ((0, 0), (pad, pad), (pad, pad), (0, 0)))
    xf = _split_phases(xp, s, flen)
    args = [xf, w, scale.astype(jnp.float32).reshape(1, Cout),
            shift.astype(jnp.float32).reshape(1, Cout)]
    in_specs = [pl.BlockSpec((1, P, flen, Cin), lambda b: (b, 0, 0, 0)),
                pl.BlockSpec((T, Cin, Cout), lambda b: (0, 0, 0)),
                pl.BlockSpec((1, Cout), lambda b: (0, 0)),
                pl.BlockSpec((1, Cout), lambda b: (0, 0))]
    out_shapes = [jax.ShapeDtypeStruct((B, L, Cout), jnp.bfloat16)]
    out_specs = [pl.BlockSpec((1, L, Cout), lambda b: (b, 0, 0))]
    if has_down:
        dw, dsc, dsh = down
        args += [dw, dsc.astype(jnp.float32).reshape(1, Cout),
                 dsh.astype(jnp.float32).reshape(1, Cout)]
        in_specs += [pl.BlockSpec((1, Cin, Cout), lambda b: (0, 0, 0)),
                     pl.BlockSpec((1, Cout), lambda b: (0, 0)),
                     pl.BlockSpec((1, Cout), lambda b: (0, 0))]
        out_shapes.append(jax.ShapeDtypeStruct((B, L, Cout), jnp.bfloat16))
        out_specs.append(pl.BlockSpec((1, L, Cout), lambda b: (b, 0, 0)))
    if has_res:
        rw = jnp.pad(residual.astype(jnp.bfloat16),
                     ((0, 0), (0, 0), (0, Wq - Wo), (0, 0)))
        rw = rw.reshape(B, Ho * Wq, Cout)
        if L > Ho * Wq:
            rw = jnp.pad(rw, ((0, 0), (0, L - Ho * Wq), (0, 0)))
        args.append(rw)
        in_specs.append(pl.BlockSpec((1, L, Cout), lambda b: (b, 0, 0)))
    outs = pl.pallas_call(
        functools.partial(_conv_body, taps=taps, relu=relu,
                          has_res=has_res, has_down=has_down),
        out_shape=out_shapes,
        grid=(B,),
        in_specs=in_specs,
        out_specs=out_specs,
        scratch_shapes=[pltpu.VMEM((L, Cout), jnp.float32)],
        compiler_params=pltpu.CompilerParams(
            dimension_semantics=("parallel",),
            vmem_limit_bytes=48 * 1024 * 1024),
    )(*args)

    def crop(o):
        return o[:, :Ho * Wq, :].reshape(B, Ho, Wq, Cout)[:, :, :Wo, :]
    if has_down:
        return crop(outs[0]), crop(outs[1])
    return crop(outs[0])


def _pool_body(x_ref, o_ref, *, taps):
    L = o_ref.shape[1]
    m = x_ref[0, taps[0][0], taps[0][1]:taps[0][1] + L, :]
    for p, off in taps[1:]:
        m = jnp.maximum(m, x_ref[0, p, off:off + L, :])
    o_ref[0] = m


def _maxpool3x3(x):
    B, H, W, C = x.shape
    Hp, Wp = H + 2, W + 2
    Ho, Wo = (Hp - 3) // 2 + 1, (Wp - 3) // 2 + 1
    Hq, Wq = _ru(Hp, 2) // 2, _ru(Wp, 2) // 2
    L = _ru(Ho * Wq, 8)
    taps = tuple(((di % 2) * 2 + (dj % 2), (di // 2) * Wq + (dj // 2))
                 for di in range(3) for dj in range(3))
    flen = max(max(off for _, off in taps) + L, Hq * Wq)
    xp = jnp.pad(x, ((0, 0), (1, 1), (1, 1), (0, 0)))
    xf = _split_phases(xp, 2, flen)
    out = pl.pallas_call(
        functools.partial(_pool_body, taps=taps),
        out_shape=jax.ShapeDtypeStruct((B, L, C), x.dtype),
        grid=(B,),
        in_specs=[pl.BlockSpec((1, 4, flen, C), lambda b: (b, 0, 0, 0))],
        out_specs=pl.BlockSpec((1, L, C), lambda b: (b, 0, 0)),
        compiler_params=pltpu.CompilerParams(
            dimension_semantics=("parallel",)),
    )(xf)
    return out[:, :Ho * Wq, :].reshape(B, Ho, Wq, C)[:, :, :Wo, :]


def _spp_body(x_ref, *o_refs, dims):
    x = x_ref[0]
    for o_ref, (kh, kw, sh, sw, oh, ow) in zip(o_refs, dims):
        vals = []
        for i in range(oh):
            for j in range(ow):
                vals.append(jnp.max(x[i * sh:i * sh + kh, j * sw:j * sw + kw, :],
                                    axis=(0, 1)))
        o_ref[0] = jnp.stack(vals, axis=0)


def _spp_feats(x, sides):
    B, H, W, C = x.shape
    dims, out_shapes, out_specs = [], [], []
    for n in sides:
        kh, kw = math.ceil(H / n), math.ceil(W / n)
        sh, sw = H // n, W // n
        oh, ow = (H - kh) // sh + 1, (W - kw) // sw + 1
        dims.append((kh, kw, sh, sw, oh, ow))
        out_shapes.append(jax.ShapeDtypeStruct((B, oh * ow, C), x.dtype))
        out_specs.append(
            pl.BlockSpec((1, oh * ow, C), lambda b: (b, 0, 0)))
    outs = pl.pallas_call(
        functools.partial(_spp_body, dims=tuple(dims)),
        out_shape=out_shapes,
        grid=(B,),
        in_specs=[pl.BlockSpec((1, H, W, C), lambda b: (b, 0, 0, 0))],
        out_specs=out_specs,
        compiler_params=pltpu.CompilerParams(
            dimension_semantics=("parallel",)),
    )(x)
    feats = [jnp.transpose(o, (0, 2, 1)).reshape(B, -1) for o in outs]
    return jnp.concatenate(feats, axis=1)


def _stem(x, w_mat, scale, shift):
    B, H, W, C = x.shape
    Ho, Wo = (H + 6 - 7) // 2 + 1, (W + 6 - 7) // 2 + 1
    xp = jnp.pad(x.astype(jnp.bfloat16), ((0, 0), (3, 3), (3, 3), (0, 0)))
    sl = [xp[:, di:di + 2 * (Ho - 1) + 1:2, dj:dj + 2 * (Wo - 1) + 1:2, :]
          for di in range(7) for dj in range(7)]
    a = jnp.concatenate(sl, axis=-1).reshape(B * Ho * Wo, 49 * C)
    y = _mm_affine(a, w_mat, scale, shift, relu=True, tm=2048,
                   out_dtype=jnp.bfloat16)
    return y.reshape(B, Ho, Wo, -1)


def _basic(h, c1, s1, b1, c2, s2, b2, stride, down):
    if down is not None:
        y, idn = _conv_bn_act(h, c1, s1, b1, k=3, stride=stride, pad=1,
                              relu=True, down=down)
    else:
        y = _conv_bn_act(h, c1, s1, b1, k=3, stride=stride, pad=1, relu=True)
        idn = h
    return _conv_bn_act(y, c2, s2, b2, k=3, stride=1, pad=1, relu=True,
                        residual=idn)


def kernel(x, conv1, bn1_scale, bn1_shift, l0b0_conv1, l0b0_bn1_scale, l0b0_bn1_shift, l0b0_conv2, l0b0_bn2_scale, l0b0_bn2_shift, l0b1_conv1, l0b1_bn1_scale, l0b1_bn1_shift, l0b1_conv2, l0b1_bn2_scale, l0b1_bn2_shift, l1b0_conv1, l1b0_bn1_scale, l1b0_bn1_shift, l1b0_conv2, l1b0_bn2_scale, l1b0_bn2_shift, l1b0_down_conv, l1b0_down_bn_scale, l1b0_down_bn_shift, l1b1_conv1, l1b1_bn1_scale, l1b1_bn1_shift, l1b1_conv2, l1b1_bn2_scale, l1b1_bn2_shift, l2b0_conv1, l2b0_bn1_scale, l2b0_bn1_shift, l2b0_conv2, l2b0_bn2_scale, l2b0_bn2_shift, l2b0_down_conv, l2b0_down_bn_scale, l2b0_down_bn_shift, l2b1_conv1, l2b1_bn1_scale, l2b1_bn1_shift, l2b1_conv2, l2b1_bn2_scale, l2b1_bn2_shift, l3b0_conv1, l3b0_bn1_scale, l3b0_bn1_shift, l3b0_conv2, l3b0_bn2_scale, l3b0_bn2_shift, l3b0_down_conv, l3b0_down_bn_scale, l3b0_down_bn_shift, l3b1_conv1, l3b1_bn1_scale, l3b1_bn1_shift, l3b1_conv2, l3b1_bn2_scale, l3b1_bn2_shift, cls_w, cls_b):
    h = jnp.transpose(x, (0, 2, 3, 1))
    h = _stem(h, conv1, bn1_scale, bn1_shift)
    h = _maxpool3x3(h)
    h = _basic(h, l0b0_conv1, l0b0_bn1_scale, l0b0_bn1_shift,
               l0b0_conv2, l0b0_bn2_scale, l0b0_bn2_shift, 1, None)
    h = _basic(h, l0b1_conv1, l0b1_bn1_scale, l0b1_bn1_shift,
               l0b1_conv2, l0b1_bn2_scale, l0b1_bn2_shift, 1, None)
    h = _basic(h, l1b0_conv1, l1b0_bn1_scale, l1b0_bn1_shift,
               l1b0_conv2, l1b0_bn2_scale, l1b0_bn2_shift, 2,
               (l1b0_down_conv, l1b0_down_bn_scale, l1b0_down_bn_shift))
    h = _basic(h, l1b1_conv1, l1b1_bn1_scale, l1b1_bn1_shift,
               l1b1_conv2, l1b1_bn2_scale, l1b1_bn2_shift, 1, None)
    h = _basic(h, l2b0_conv1, l2b0_bn1_scale, l2b0_bn1_shift,
               l2b0_conv2, l2b0_bn2_scale, l2b0_bn2_shift, 2,
               (l2b0_down_conv, l2b0_down_bn_scale, l2b0_down_bn_shift))
    h = _basic(h, l2b1_conv1, l2b1_bn1_scale, l2b1_bn1_shift,
               l2b1_conv2, l2b1_bn2_scale, l2b1_bn2_shift, 1, None)
    h = _basic(h, l3b0_conv1, l3b0_bn1_scale, l3b0_bn1_shift,
               l3b0_conv2, l3b0_bn2_scale, l3b0_bn2_shift, 2,
               (l3b0_down_conv, l3b0_down_bn_scale, l3b0_down_bn_shift))
    h = _basic(h, l3b1_conv1, l3b1_bn1_scale, l3b1_bn1_shift,
               l3b1_conv2, l3b1_bn2_scale, l3b1_bn2_shift, 1, None)
    feats = _spp_feats(h, (1, 2, 6))
    logits = _mm_affine(feats, cls_w, jnp.ones_like(cls_b), cls_b,
                        relu=False, tm=32, out_dtype=jnp.float32)
    return logits
